# baseline (speedup 1.0000x reference)
import functools

import jax
import jax.numpy as jnp
from jax import lax
from jax.experimental import pallas as pl
from jax.experimental.pallas import tpu as pltpu
from jax.experimental.pallas import tpu_sc as plsc

DIAM = 512
EDIM = 64


def _renorm_body(t_ref, o_ref):
    t = t_ref[...]
    norms = jnp.max(jnp.abs(t), axis=1, keepdims=True)
    scale = jnp.where(norms > 1.0, 1.0 / (norms + 1e-7), 1.0)
    o_ref[...] = t * scale


def _renorm(table):
    return pl.pallas_call(
        _renorm_body,
        out_shape=jax.ShapeDtypeStruct(table.shape, table.dtype),
    )(table)


def _sc_gather(idx_flat, table):
    B = idx_flat.shape[0]
    NW = 32
    CH = 128
    b_per_w = B // NW
    n_ch = b_per_w // CH
    assert b_per_w * NW == B and n_ch * CH == b_per_w

    mesh = plsc.VectorSubcoreMesh(core_axis_name="c", subcore_axis_name="s")

    @functools.partial(
        pl.kernel,
        mesh=mesh,
        compiler_params=pltpu.CompilerParams(use_tc_tiling_on_sc=False),
        out_type=jax.ShapeDtypeStruct((B, EDIM), jnp.float32),
        scratch_types=[
            pltpu.VMEM((CH,), jnp.int32),
            pltpu.VMEM((CH, EDIM), jnp.float32),
            pltpu.SemaphoreType.DMA,
        ],
    )
    def k(idx_hbm, tbl_hbm, out_hbm, idx_v, rows_v, sem):
        wid = lax.axis_index("s") * 2 + lax.axis_index("c")
        base = wid * b_per_w

        def body(j, carry):
            off = base + j * CH
            pltpu.sync_copy(idx_hbm.at[pl.ds(off, CH)], idx_v)
            for i in range(CH // 16):
                sl = pl.ds(i * 16, 16)
                idx_v[sl] = jnp.minimum(idx_v[sl], DIAM)
            pltpu.async_copy(tbl_hbm.at[idx_v], rows_v, sem).wait()
            pltpu.sync_copy(rows_v, out_hbm.at[pl.ds(off, CH)])
            return carry

        lax.fori_loop(0, n_ch, body, 0)

    return k(idx_flat, table)


def kernel(x, table):
    renormed = _renorm(table)
    out = _sc_gather(x.reshape(-1), renormed)
    return out.reshape(x.shape + (EDIM,))
